# baseline (speedup 1.0000x reference)
import jax
import jax.numpy as jnp
from jax import lax
from jax.experimental import pallas as pl
from jax.experimental.pallas import tpu as pltpu
from jax.experimental.pallas import tpu_sc as plsc

G = 7
NBINS = G * G
D = 21
NC = D * NBINS
DPAD = 22
DH = DPAD // 2
SCALE = 0.0625
H = 64
W = 64
WPAD = 72
PLANE_HW = (H + 1) * WPAD
PLANE_WORDS = DPAD * PLANE_HW
HALF_WORDS = DH * PLANE_HW
UNIT_WORDS = 2 * HALF_WORDS
NROI = 5000
NROI_PAD = 5120
NCHUNKS = 20
GPC = 16
RPP = 32
PIECE_WORDS = RPP * DH
STAGE_WORDS = GPC * 16 * DH
PPC = (GPC * 16) // RPP
NUNITS = 2 * NBINS
NWORKERS = 32
UNITS_PER_TILE = 4
QCHUNK_WORDS = NUNITS * PIECE_WORDS
NQ = NROI_PAD // RPP
NQ_T = 157
QPT = 5
TAIL_ROIS = NROI - (NQ_T - 1) * RPP


def _integral_tc_kernel(f_ref, o_ref):
    row = lax.broadcasted_iota(jnp.int32, (H, H), 0)
    col = lax.broadcasted_iota(jnp.int32, (H, H), 1)
    lower = (row >= col).astype(jnp.float32)
    upper = (row <= col).astype(jnp.float32)
    for d in range(D):
        f = f_ref[0, d, 0]
        a = jnp.dot(lower, f, preferred_element_type=jnp.float32,
                    precision=lax.Precision.HIGHEST)
        b = jnp.dot(a, upper, preferred_element_type=jnp.float32,
                    precision=lax.Precision.HIGHEST)
        buf = jnp.concatenate(
            [jnp.zeros((H, 1), jnp.float32), b,
             jnp.zeros((H, WPAD - 1 - W), jnp.float32)], axis=1)
        buf = jnp.concatenate([jnp.zeros((1, WPAD), jnp.float32), buf],
                              axis=0)
        o_ref[0, 0, d] = buf
    o_ref[0, 0, D] = jnp.zeros((H + 1, WPAD), jnp.float32)


def _integral_image(feat5):
    return pl.pallas_call(
        _integral_tc_kernel,
        grid=(2, NBINS),
        in_specs=[pl.BlockSpec((1, D, 1, H, W), lambda b, p: (b, 0, p, 0, 0))],
        out_specs=pl.BlockSpec((1, 1, DPAD, H + 1, WPAD),
                               lambda b, p: (b, p, 0, 0, 0)),
        out_shape=jax.ShapeDtypeStruct((2, NBINS, DPAD, H + 1, WPAD),
                                       jnp.float32),
    )(feat5)


def _bin_bounds(rois):
    pf = jnp.arange(G, dtype=jnp.float32)[None, :]
    rsw = (jnp.round(rois[:, 1]) * SCALE)[:, None]
    rsh = (jnp.round(rois[:, 2]) * SCALE)[:, None]
    rew = (jnp.round(rois[:, 3] + 1.0) * SCALE)[:, None]
    reh = (jnp.round(rois[:, 4] + 1.0) * SCALE)[:, None]
    roi_w = jnp.maximum(rew - rsw, 0.1)
    roi_h = jnp.maximum(reh - rsh, 0.1)
    bsh = roi_h / G
    bsw = roi_w / G
    hs = jnp.clip(jnp.floor(pf * bsh + rsh), 0, H).astype(jnp.int32)
    he = jnp.clip(jnp.ceil((pf + 1.0) * bsh + rsh), 0, H).astype(jnp.int32)
    ws = jnp.clip(jnp.floor(pf * bsw + rsw), 0, W).astype(jnp.int32)
    we = jnp.clip(jnp.ceil((pf + 1.0) * bsw + rsw), 0, W).astype(jnp.int32)
    return hs, he, ws, we


def _pool_sc_kernel(i_hbm, tb_hbm, wswe_hbm, inv_hbm,
                    out_hbm, unit_v, tb_v, wswe_v, inv_v, stage_v, sem):
    wid = lax.axis_index("s") * 2 + lax.axis_index("c")
    lanes = lax.iota(jnp.int32, 16)
    lanes_d = lanes * DH

    def process_unit(u):
        binidx = u // 2
        dh = u - binidx * 2
        ph = binidx // G
        pw = binidx - ph * G
        doff = dh * HALF_WORDS
        for b in range(2):
            src = (b * NBINS + binidx) * PLANE_WORDS + doff
            pltpu.sync_copy(
                i_hbm.at[pl.ds(src, HALF_WORDS)],
                unit_v.at[pl.ds(b * HALF_WORDS, HALF_WORDS)])
        pltpu.sync_copy(tb_hbm.at[ph], tb_v)
        pltpu.sync_copy(wswe_hbm.at[pw], wswe_v)
        pltpu.sync_copy(inv_hbm.at[binidx], inv_v)
        unit_off = binidx * (2 * PIECE_WORDS) + dh * PIECE_WORDS

        def chunk_body(c, carry):
            def group_body(j, carry2):
                base = (c * GPC + j) * 16
                xt = tb_v[pl.ds(base, 16)]
                xw = wswe_v[pl.ds(base, 16)]
                inv = inv_v[pl.ds(base, 16)]
                top = xt & 0xFFFF
                bot = (xt >> 16) & 0xFFFF
                ws = xw & 0xFFFF
                we = xw >> 16
                i_ee = bot + we
                i_se = top + we
                i_es = bot + ws
                i_ss = top + ws
                sbase = j * (16 * DH) + lanes_d
                for d in range(DH):
                    sub = unit_v.at[pl.ds(d * PLANE_HW,
                                          UNIT_WORDS - d * PLANE_HW)]
                    g1 = plsc.load_gather(sub, [i_ee])
                    g2 = plsc.load_gather(sub, [i_se])
                    g3 = plsc.load_gather(sub, [i_es])
                    g4 = plsc.load_gather(sub, [i_ss])
                    val = (g1 - g2 - g3 + g4) * inv
                    plsc.store_scatter(stage_v, [sbase + d], val)
                return carry2

            lax.fori_loop(0, GPC, group_body, 0, unroll=2)
            copies = []
            for i in range(PPC):
                q = c * PPC + i
                copies.append(pltpu.async_copy(
                    stage_v.at[pl.ds(i * PIECE_WORDS, PIECE_WORDS)],
                    out_hbm.at[pl.ds(q * QCHUNK_WORDS + unit_off,
                                     PIECE_WORDS)],
                    sem))
            for cp in copies:
                cp.wait()
            return carry

        lax.fori_loop(0, NCHUNKS, chunk_body, 0)

    def unit_body(k, carry):
        u = wid + k * NWORKERS

        @pl.when(u < NUNITS)
        def _():
            process_unit(u)

        return carry

    lax.fori_loop(0, UNITS_PER_TILE, unit_body, 0)


def _pool(i_img, tb, wswe, inv):
    mesh = plsc.VectorSubcoreMesh(core_axis_name="c", subcore_axis_name="s")
    f = pl.kernel(
        _pool_sc_kernel,
        out_type=jax.ShapeDtypeStruct((NQ * QCHUNK_WORDS,), jnp.float32),
        mesh=mesh,
        compiler_params=pltpu.CompilerParams(needs_layout_passes=False),
        scratch_types=[
            pltpu.VMEM((UNIT_WORDS,), jnp.float32),
            pltpu.VMEM((NROI_PAD,), jnp.int32),
            pltpu.VMEM((NROI_PAD,), jnp.int32),
            pltpu.VMEM((NROI_PAD,), jnp.float32),
            pltpu.VMEM((STAGE_WORDS,), jnp.float32),
            pltpu.SemaphoreType.DMA,
        ],
    )
    return f(i_img, tb, wswe, inv)


def _transpose_sc_kernel(pool_hbm, out_hbm, in_v, obuf_v, sem):
    wid = lax.axis_index("s") * 2 + lax.axis_index("c")
    lanes = lax.iota(jnp.int32, 16)

    def process_chunk(q):
        pltpu.sync_copy(pool_hbm.at[pl.ds(q * QCHUNK_WORDS, QCHUNK_WORDS)],
                        in_v)
        boffs = []
        osels = []
        for half in range(RPP // 16):
            r_vec = lanes + half * 16
            boffs.append(r_vec * DH)
            osels.append(r_vec)

        def c_body(c, carry):
            binidx = c % NBINS
            d = c // NBINS
            dh = d // DH
            dl = d - dh * DH
            pre_c = binidx * (2 * PIECE_WORDS) + dh * PIECE_WORDS + dl
            cvec = jnp.full((16,), c, jnp.int32)
            for half in range(RPP // 16):
                v = plsc.load_gather(in_v, [boffs[half] + pre_c])
                plsc.store_scatter(obuf_v, [osels[half], cvec], v)
            return carry

        lax.fori_loop(0, NC, c_body, 0, unroll=4)

        @pl.when(q < NQ_T - 1)
        def _():
            pltpu.sync_copy(obuf_v, out_hbm.at[pl.ds(q * RPP, RPP)])

        @pl.when(q == NQ_T - 1)
        def _():
            pltpu.sync_copy(
                obuf_v.at[pl.ds(0, TAIL_ROIS)],
                out_hbm.at[pl.ds(q * RPP, TAIL_ROIS)])

    def chunk_loop(k, carry):
        q = wid + k * NWORKERS

        @pl.when(q < NQ_T)
        def _():
            process_chunk(q)

        return carry

    lax.fori_loop(0, QPT, chunk_loop, 0)


def _transpose(pool_out):
    mesh = plsc.VectorSubcoreMesh(core_axis_name="c", subcore_axis_name="s")
    f = pl.kernel(
        _transpose_sc_kernel,
        out_type=jax.ShapeDtypeStruct((NROI, NC), jnp.float32),
        mesh=mesh,
        compiler_params=pltpu.CompilerParams(needs_layout_passes=False),
        scratch_types=[
            pltpu.VMEM((QCHUNK_WORDS,), jnp.float32),
            pltpu.VMEM((RPP, NC), jnp.float32),
            pltpu.SemaphoreType.DMA,
        ],
    )
    return f(pool_out)


def kernel(features, rois):
    feat5 = features.reshape(2, D, NBINS, H, W)
    i_img = _integral_image(feat5).reshape(NUNITS * PLANE_WORDS)
    hs, he, ws, we = _bin_bounds(rois)
    pad_b = jnp.zeros((NROI_PAD - NROI, G), jnp.int32)

    def padt(x):
        return jnp.concatenate([x, pad_b], axis=0).T

    bterm = (rois[:, 0].astype(jnp.int32) * HALF_WORDS)[:, None]
    tb = padt((hs * WPAD + bterm) | ((he * WPAD + bterm) << 16))
    wswe = padt(ws | (we << 16))
    area = ((he - hs).astype(jnp.float32)[:, :, None]
            * (we - ws).astype(jnp.float32)[:, None, :])
    empty = (he <= hs)[:, :, None] | (we <= ws)[:, None, :]
    inv = jnp.where(empty, 0.0, 1.0 / jnp.maximum(area, 1.0))
    inv = jnp.concatenate(
        [inv.reshape(NROI, NBINS),
         jnp.zeros((NROI_PAD - NROI, NBINS), jnp.float32)], axis=0).T
    pool_out = _pool(i_img, tb, wswe, inv)
    return _transpose(pool_out).reshape(NROI, D, G, G)


# speedup vs baseline: 1.0020x; 1.0020x over previous
import jax
import jax.numpy as jnp
from jax import lax
from jax.experimental import pallas as pl
from jax.experimental.pallas import tpu as pltpu
from jax.experimental.pallas import tpu_sc as plsc

G = 7
NBINS = G * G
D = 21
NC = D * NBINS
DPAD = 22
DH = DPAD // 2
SCALE = 0.0625
H = 64
W = 64
WPAD = 72
PLANE_HW = (H + 1) * WPAD
PLANE_WORDS = DPAD * PLANE_HW
HALF_WORDS = DH * PLANE_HW
UNIT_WORDS = 2 * HALF_WORDS
NROI = 5000
NROI_PAD = 5120
NCHUNKS = 20
GPC = 16
RPP = 32
PIECE_WORDS = RPP * DH
STAGE_WORDS = GPC * 16 * DH
PPC = (GPC * 16) // RPP
NUNITS = 2 * NBINS
NWORKERS = 32
UNITS_PER_TILE = 4
QCHUNK_WORDS = NUNITS * PIECE_WORDS
NQ = NROI_PAD // RPP
NQ_T = 157
QPT = 5
TAIL_ROIS = NROI - (NQ_T - 1) * RPP


def _integral_tc_kernel(f_ref, o_ref):
    row = lax.broadcasted_iota(jnp.int32, (H, H), 0)
    col = lax.broadcasted_iota(jnp.int32, (H, H), 1)
    lower = (row >= col).astype(jnp.float32)
    upper = (row <= col).astype(jnp.float32)
    for d in range(D):
        f = f_ref[0, d, 0]
        a = jnp.dot(lower, f, preferred_element_type=jnp.float32,
                    precision=lax.Precision.HIGHEST)
        b = jnp.dot(a, upper, preferred_element_type=jnp.float32,
                    precision=lax.Precision.HIGHEST)
        buf = jnp.concatenate(
            [jnp.zeros((H, 1), jnp.float32), b,
             jnp.zeros((H, WPAD - 1 - W), jnp.float32)], axis=1)
        buf = jnp.concatenate([jnp.zeros((1, WPAD), jnp.float32), buf],
                              axis=0)
        o_ref[0, 0, d] = buf
    o_ref[0, 0, D] = jnp.zeros((H + 1, WPAD), jnp.float32)


def _integral_image(feat5):
    return pl.pallas_call(
        _integral_tc_kernel,
        grid=(2, NBINS),
        in_specs=[pl.BlockSpec((1, D, 1, H, W), lambda b, p: (b, 0, p, 0, 0))],
        out_specs=pl.BlockSpec((1, 1, DPAD, H + 1, WPAD),
                               lambda b, p: (b, p, 0, 0, 0)),
        out_shape=jax.ShapeDtypeStruct((2, NBINS, DPAD, H + 1, WPAD),
                                       jnp.float32),
    )(feat5)


def _bin_bounds(rois):
    pf = jnp.arange(G, dtype=jnp.float32)[None, :]
    rsw = (jnp.round(rois[:, 1]) * SCALE)[:, None]
    rsh = (jnp.round(rois[:, 2]) * SCALE)[:, None]
    rew = (jnp.round(rois[:, 3] + 1.0) * SCALE)[:, None]
    reh = (jnp.round(rois[:, 4] + 1.0) * SCALE)[:, None]
    roi_w = jnp.maximum(rew - rsw, 0.1)
    roi_h = jnp.maximum(reh - rsh, 0.1)
    bsh = roi_h / G
    bsw = roi_w / G
    hs = jnp.clip(jnp.floor(pf * bsh + rsh), 0, H).astype(jnp.int32)
    he = jnp.clip(jnp.ceil((pf + 1.0) * bsh + rsh), 0, H).astype(jnp.int32)
    ws = jnp.clip(jnp.floor(pf * bsw + rsw), 0, W).astype(jnp.int32)
    we = jnp.clip(jnp.ceil((pf + 1.0) * bsw + rsw), 0, W).astype(jnp.int32)
    return hs, he, ws, we


def _pool_sc_kernel(i_hbm, hshe_hbm, wswe_hbm, batch_hbm,
                    out_hbm, unit_v, hshe_v, wswe_v, batch_v, stage_v, sem):
    wid = lax.axis_index("s") * 2 + lax.axis_index("c")
    lanes = lax.iota(jnp.int32, 16)
    lanes_d = lanes * DH
    pltpu.sync_copy(batch_hbm, batch_v)

    def process_unit(u):
        binidx = u // 2
        dh = u - binidx * 2
        ph = binidx // G
        pw = binidx - ph * G
        doff = dh * HALF_WORDS
        for b in range(2):
            src = (b * NBINS + binidx) * PLANE_WORDS + doff
            pltpu.sync_copy(
                i_hbm.at[pl.ds(src, HALF_WORDS)],
                unit_v.at[pl.ds(b * HALF_WORDS, HALF_WORDS)])
        pltpu.sync_copy(hshe_hbm.at[ph], hshe_v)
        pltpu.sync_copy(wswe_hbm.at[pw], wswe_v)
        unit_off = binidx * (2 * PIECE_WORDS) + dh * PIECE_WORDS

        def chunk_body(c, carry):
            def group_body(j, carry2):
                base = (c * GPC + j) * 16
                xh = hshe_v[pl.ds(base, 16)]
                xw = wswe_v[pl.ds(base, 16)]
                bvec = batch_v[pl.ds(base, 16)]
                hs = xh & 0xFFFF
                he = xh >> 16
                ws = xw & 0xFFFF
                we = xw >> 16
                area = ((he - hs) * (we - ws)).astype(jnp.float32)
                empty = (he <= hs) | (we <= ws)
                inv = jnp.where(empty, 0.0, 1.0 / jnp.maximum(area, 1.0))
                bterm = bvec * HALF_WORDS
                top = hs * WPAD + bterm
                bot = he * WPAD + bterm
                i_ee = bot + we
                i_se = top + we
                i_es = bot + ws
                i_ss = top + ws
                sbase = j * (16 * DH) + lanes_d
                for d in range(DH):
                    sub = unit_v.at[pl.ds(d * PLANE_HW,
                                          UNIT_WORDS - d * PLANE_HW)]
                    g1 = plsc.load_gather(sub, [i_ee])
                    g2 = plsc.load_gather(sub, [i_se])
                    g3 = plsc.load_gather(sub, [i_es])
                    g4 = plsc.load_gather(sub, [i_ss])
                    val = (g1 - g2 - g3 + g4) * inv
                    plsc.store_scatter(stage_v, [sbase + d], val)
                return carry2

            lax.fori_loop(0, GPC, group_body, 0, unroll=2)
            copies = []
            for i in range(PPC):
                q = c * PPC + i
                copies.append(pltpu.async_copy(
                    stage_v.at[pl.ds(i * PIECE_WORDS, PIECE_WORDS)],
                    out_hbm.at[pl.ds(q * QCHUNK_WORDS + unit_off,
                                     PIECE_WORDS)],
                    sem))
            for cp in copies:
                cp.wait()
            return carry

        lax.fori_loop(0, NCHUNKS, chunk_body, 0)

    def unit_body(k, carry):
        u = wid + k * NWORKERS

        @pl.when(u < NUNITS)
        def _():
            process_unit(u)

        return carry

    lax.fori_loop(0, UNITS_PER_TILE, unit_body, 0)


def _pool(i_img, hshe, wswe, batch):
    mesh = plsc.VectorSubcoreMesh(core_axis_name="c", subcore_axis_name="s")
    f = pl.kernel(
        _pool_sc_kernel,
        out_type=jax.ShapeDtypeStruct((NQ * QCHUNK_WORDS,), jnp.float32),
        mesh=mesh,
        compiler_params=pltpu.CompilerParams(needs_layout_passes=False),
        scratch_types=[
            pltpu.VMEM((UNIT_WORDS,), jnp.float32),
            pltpu.VMEM((NROI_PAD,), jnp.int32),
            pltpu.VMEM((NROI_PAD,), jnp.int32),
            pltpu.VMEM((NROI_PAD,), jnp.int32),
            pltpu.VMEM((STAGE_WORDS,), jnp.float32),
            pltpu.SemaphoreType.DMA,
        ],
    )
    return f(i_img, hshe, wswe, batch)


def _transpose_sc_kernel(pool_hbm, out_hbm, in_v, obuf_v, sem):
    wid = lax.axis_index("s") * 2 + lax.axis_index("c")
    lanes = lax.iota(jnp.int32, 16)

    def process_chunk(q):
        pltpu.sync_copy(pool_hbm.at[pl.ds(q * QCHUNK_WORDS, QCHUNK_WORDS)],
                        in_v)
        boffs = []
        osels = []
        for half in range(RPP // 16):
            r_vec = lanes + half * 16
            boffs.append(r_vec * DH)
            osels.append(r_vec)

        def c_body(c, carry):
            binidx = c % NBINS
            d = c // NBINS
            dh = d // DH
            dl = d - dh * DH
            pre_c = binidx * (2 * PIECE_WORDS) + dh * PIECE_WORDS + dl
            cvec = jnp.full((16,), c, jnp.int32)
            for half in range(RPP // 16):
                v = plsc.load_gather(in_v, [boffs[half] + pre_c])
                plsc.store_scatter(obuf_v, [osels[half], cvec], v)
            return carry

        lax.fori_loop(0, NC, c_body, 0, unroll=4)

        @pl.when(q < NQ_T - 1)
        def _():
            pltpu.sync_copy(obuf_v, out_hbm.at[pl.ds(q * RPP, RPP)])

        @pl.when(q == NQ_T - 1)
        def _():
            pltpu.sync_copy(
                obuf_v.at[pl.ds(0, TAIL_ROIS)],
                out_hbm.at[pl.ds(q * RPP, TAIL_ROIS)])

    def chunk_loop(k, carry):
        q = wid + k * NWORKERS

        @pl.when(q < NQ_T)
        def _():
            process_chunk(q)

        return carry

    lax.fori_loop(0, QPT, chunk_loop, 0)


def _transpose(pool_out):
    mesh = plsc.VectorSubcoreMesh(core_axis_name="c", subcore_axis_name="s")
    f = pl.kernel(
        _transpose_sc_kernel,
        out_type=jax.ShapeDtypeStruct((NROI, NC), jnp.float32),
        mesh=mesh,
        compiler_params=pltpu.CompilerParams(needs_layout_passes=False),
        scratch_types=[
            pltpu.VMEM((QCHUNK_WORDS,), jnp.float32),
            pltpu.VMEM((RPP, NC), jnp.float32),
            pltpu.SemaphoreType.DMA,
        ],
    )
    return f(pool_out)


def kernel(features, rois):
    feat5 = features.reshape(2, D, NBINS, H, W)
    i_img = _integral_image(feat5).reshape(NUNITS * PLANE_WORDS)
    hs, he, ws, we = _bin_bounds(rois)
    pad_b = jnp.zeros((NROI_PAD - NROI, G), jnp.int32)

    def padt(x):
        return jnp.concatenate([x, pad_b], axis=0).T

    hshe = padt(hs | (he << 16))
    wswe = padt(ws | (we << 16))
    batch = jnp.concatenate(
        [rois[:, 0].astype(jnp.int32),
         jnp.zeros((NROI_PAD - NROI,), jnp.int32)])
    pool_out = _pool(i_img, hshe, wswe, batch)
    return _transpose(pool_out).reshape(NROI, D, G, G)


# speedup vs baseline: 1.1646x; 1.1623x over previous
import jax
import jax.numpy as jnp
from jax import lax
from jax.experimental import pallas as pl
from jax.experimental.pallas import tpu as pltpu
from jax.experimental.pallas import tpu_sc as plsc

G = 7
NBINS = G * G
D = 21
NC = D * NBINS
DPAD = 22
DH = DPAD // 2
SCALE = 0.0625
H = 64
W = 64
WPAD = 72
PLANE_HW = (H + 1) * WPAD
PLANE_WORDS = DPAD * PLANE_HW
HALF_WORDS = DH * PLANE_HW
UNIT_WORDS = 2 * HALF_WORDS
NROI = 5000
NROI_PAD = 5120
NCHUNKS = 20
GPC = 16
RPP = 32
PIECE_WORDS = RPP * DH
STAGE_WORDS = GPC * 16 * DH
PPC = (GPC * 16) // RPP
NUNITS = 2 * NBINS
NWORKERS = 32
UNITS_PER_TILE = 4
QCHUNK_WORDS = NUNITS * PIECE_WORDS
NQ = NROI_PAD // RPP
NQ_T = 157
QPT = 5
TAIL_ROIS = NROI - (NQ_T - 1) * RPP


def _integral_tc_kernel(f_ref, o_ref):
    row = lax.broadcasted_iota(jnp.int32, (H, H), 0)
    col = lax.broadcasted_iota(jnp.int32, (H, H), 1)
    lower = (row >= col).astype(jnp.float32)
    upper = (row <= col).astype(jnp.float32)
    for d in range(D):
        f = f_ref[0, d, 0]
        a = jnp.dot(lower, f, preferred_element_type=jnp.float32,
                    precision=lax.Precision.HIGHEST)
        b = jnp.dot(a, upper, preferred_element_type=jnp.float32,
                    precision=lax.Precision.HIGHEST)
        buf = jnp.concatenate(
            [jnp.zeros((H, 1), jnp.float32), b,
             jnp.zeros((H, WPAD - 1 - W), jnp.float32)], axis=1)
        buf = jnp.concatenate([jnp.zeros((1, WPAD), jnp.float32), buf],
                              axis=0)
        o_ref[0, 0, d] = buf
    o_ref[0, 0, D] = jnp.zeros((H + 1, WPAD), jnp.float32)


def _integral_image(feat5):
    return pl.pallas_call(
        _integral_tc_kernel,
        grid=(2, NBINS),
        in_specs=[pl.BlockSpec((1, D, 1, H, W), lambda b, p: (b, 0, p, 0, 0))],
        out_specs=pl.BlockSpec((1, 1, DPAD, H + 1, WPAD),
                               lambda b, p: (b, p, 0, 0, 0)),
        out_shape=jax.ShapeDtypeStruct((2, NBINS, DPAD, H + 1, WPAD),
                                       jnp.float32),
    )(feat5)


def _bin_bounds(rois):
    pf = jnp.arange(G, dtype=jnp.float32)[None, :]
    rsw = (jnp.round(rois[:, 1]) * SCALE)[:, None]
    rsh = (jnp.round(rois[:, 2]) * SCALE)[:, None]
    rew = (jnp.round(rois[:, 3] + 1.0) * SCALE)[:, None]
    reh = (jnp.round(rois[:, 4] + 1.0) * SCALE)[:, None]
    roi_w = jnp.maximum(rew - rsw, 0.1)
    roi_h = jnp.maximum(reh - rsh, 0.1)
    bsh = roi_h / G
    bsw = roi_w / G
    hs = jnp.clip(jnp.floor(pf * bsh + rsh), 0, H).astype(jnp.int32)
    he = jnp.clip(jnp.ceil((pf + 1.0) * bsh + rsh), 0, H).astype(jnp.int32)
    ws = jnp.clip(jnp.floor(pf * bsw + rsw), 0, W).astype(jnp.int32)
    we = jnp.clip(jnp.ceil((pf + 1.0) * bsw + rsw), 0, W).astype(jnp.int32)
    return hs, he, ws, we


def _pool_sc_kernel(i_hbm, hshe_hbm, wswe_hbm, batch_hbm,
                    out_hbm, unit_v, hshe_v, wswe_v, batch_v, stage_v, sem):
    wid = lax.axis_index("s") * 2 + lax.axis_index("c")
    lanes = lax.iota(jnp.int32, 16)
    lanes_d = lanes * DH
    pltpu.sync_copy(batch_hbm, batch_v)

    def process_unit(u):
        binidx = u // 2
        dh = u - binidx * 2
        ph = binidx // G
        pw = binidx - ph * G
        doff = dh * HALF_WORDS
        for b in range(2):
            src = (b * NBINS + binidx) * PLANE_WORDS + doff
            pltpu.sync_copy(
                i_hbm.at[pl.ds(src, HALF_WORDS)],
                unit_v.at[pl.ds(b * HALF_WORDS, HALF_WORDS)])
        pltpu.sync_copy(hshe_hbm.at[ph], hshe_v)
        pltpu.sync_copy(wswe_hbm.at[pw], wswe_v)
        unit_off = binidx * (2 * PIECE_WORDS) + dh * PIECE_WORDS

        def chunk_body(c, carry):
            @plsc.parallel_loop(0, GPC, unroll=2)
            def group_body(j):
                base = (c * GPC + j) * 16
                xh = hshe_v[pl.ds(base, 16)]
                xw = wswe_v[pl.ds(base, 16)]
                bvec = batch_v[pl.ds(base, 16)]
                hs = xh & 0xFFFF
                he = xh >> 16
                ws = xw & 0xFFFF
                we = xw >> 16
                area = ((he - hs) * (we - ws)).astype(jnp.float32)
                empty = (he <= hs) | (we <= ws)
                inv = jnp.where(empty, 0.0, 1.0 / jnp.maximum(area, 1.0))
                bterm = bvec * HALF_WORDS
                top = hs * WPAD + bterm
                bot = he * WPAD + bterm
                i_ee = bot + we
                i_se = top + we
                i_es = bot + ws
                i_ss = top + ws
                sbase = j * (16 * DH) + lanes_d
                for d in range(DH):
                    sub = unit_v.at[pl.ds(d * PLANE_HW,
                                          UNIT_WORDS - d * PLANE_HW)]
                    g1 = plsc.load_gather(sub, [i_ee])
                    g2 = plsc.load_gather(sub, [i_se])
                    g3 = plsc.load_gather(sub, [i_es])
                    g4 = plsc.load_gather(sub, [i_ss])
                    val = (g1 - g2 - g3 + g4) * inv
                    plsc.store_scatter(stage_v, [sbase + d], val)

            copies = []
            for i in range(PPC):
                q = c * PPC + i
                copies.append(pltpu.async_copy(
                    stage_v.at[pl.ds(i * PIECE_WORDS, PIECE_WORDS)],
                    out_hbm.at[pl.ds(q * QCHUNK_WORDS + unit_off,
                                     PIECE_WORDS)],
                    sem))
            for cp in copies:
                cp.wait()
            return carry

        lax.fori_loop(0, NCHUNKS, chunk_body, 0)

    def unit_body(k, carry):
        u = wid + k * NWORKERS

        @pl.when(u < NUNITS)
        def _():
            process_unit(u)

        return carry

    lax.fori_loop(0, UNITS_PER_TILE, unit_body, 0)


def _pool(i_img, hshe, wswe, batch):
    mesh = plsc.VectorSubcoreMesh(core_axis_name="c", subcore_axis_name="s")
    f = pl.kernel(
        _pool_sc_kernel,
        out_type=jax.ShapeDtypeStruct((NQ * QCHUNK_WORDS,), jnp.float32),
        mesh=mesh,
        compiler_params=pltpu.CompilerParams(needs_layout_passes=False),
        scratch_types=[
            pltpu.VMEM((UNIT_WORDS,), jnp.float32),
            pltpu.VMEM((NROI_PAD,), jnp.int32),
            pltpu.VMEM((NROI_PAD,), jnp.int32),
            pltpu.VMEM((NROI_PAD,), jnp.int32),
            pltpu.VMEM((STAGE_WORDS,), jnp.float32),
            pltpu.SemaphoreType.DMA,
        ],
    )
    return f(i_img, hshe, wswe, batch)


def _transpose_sc_kernel(pool_hbm, out_hbm, in_v, obuf_v, sem):
    wid = lax.axis_index("s") * 2 + lax.axis_index("c")
    lanes = lax.iota(jnp.int32, 16)

    def process_chunk(q):
        pltpu.sync_copy(pool_hbm.at[pl.ds(q * QCHUNK_WORDS, QCHUNK_WORDS)],
                        in_v)
        boffs = []
        osels = []
        for half in range(RPP // 16):
            r_vec = lanes + half * 16
            boffs.append(r_vec * DH)
            osels.append(r_vec)

        def c_body(c, carry):
            binidx = c % NBINS
            d = c // NBINS
            dh = d // DH
            dl = d - dh * DH
            pre_c = binidx * (2 * PIECE_WORDS) + dh * PIECE_WORDS + dl
            cvec = jnp.full((16,), c, jnp.int32)
            for half in range(RPP // 16):
                v = plsc.load_gather(in_v, [boffs[half] + pre_c])
                plsc.store_scatter(obuf_v, [osels[half], cvec], v)
            return carry

        lax.fori_loop(0, NC, c_body, 0, unroll=4)

        @pl.when(q < NQ_T - 1)
        def _():
            pltpu.sync_copy(obuf_v, out_hbm.at[pl.ds(q * RPP, RPP)])

        @pl.when(q == NQ_T - 1)
        def _():
            pltpu.sync_copy(
                obuf_v.at[pl.ds(0, TAIL_ROIS)],
                out_hbm.at[pl.ds(q * RPP, TAIL_ROIS)])

    def chunk_loop(k, carry):
        q = wid + k * NWORKERS

        @pl.when(q < NQ_T)
        def _():
            process_chunk(q)

        return carry

    lax.fori_loop(0, QPT, chunk_loop, 0)


def _transpose(pool_out):
    mesh = plsc.VectorSubcoreMesh(core_axis_name="c", subcore_axis_name="s")
    f = pl.kernel(
        _transpose_sc_kernel,
        out_type=jax.ShapeDtypeStruct((NROI, NC), jnp.float32),
        mesh=mesh,
        compiler_params=pltpu.CompilerParams(needs_layout_passes=False),
        scratch_types=[
            pltpu.VMEM((QCHUNK_WORDS,), jnp.float32),
            pltpu.VMEM((RPP, NC), jnp.float32),
            pltpu.SemaphoreType.DMA,
        ],
    )
    return f(pool_out)


def kernel(features, rois):
    feat5 = features.reshape(2, D, NBINS, H, W)
    i_img = _integral_image(feat5).reshape(NUNITS * PLANE_WORDS)
    hs, he, ws, we = _bin_bounds(rois)
    pad_b = jnp.zeros((NROI_PAD - NROI, G), jnp.int32)

    def padt(x):
        return jnp.concatenate([x, pad_b], axis=0).T

    hshe = padt(hs | (he << 16))
    wswe = padt(ws | (we << 16))
    batch = jnp.concatenate(
        [rois[:, 0].astype(jnp.int32),
         jnp.zeros((NROI_PAD - NROI,), jnp.int32)])
    pool_out = _pool(i_img, hshe, wswe, batch)
    return _transpose(pool_out).reshape(NROI, D, G, G)


# speedup vs baseline: 1.1838x; 1.0165x over previous
import jax
import jax.numpy as jnp
from jax import lax
from jax.experimental import pallas as pl
from jax.experimental.pallas import tpu as pltpu
from jax.experimental.pallas import tpu_sc as plsc

G = 7
NBINS = G * G
D = 21
NC = D * NBINS
DPAD = 22
DH = DPAD // 2
SCALE = 0.0625
H = 64
W = 64
WPAD = 72
PLANE_HW = (H + 1) * WPAD
PLANE_WORDS = DPAD * PLANE_HW
HALF_WORDS = DH * PLANE_HW
UNIT_WORDS = 2 * HALF_WORDS
NROI = 5000
NROI_PAD = 5120
NCHUNKS = 20
GPC = 16
RPP = 32
PIECE_WORDS = RPP * DH
STAGE_WORDS = GPC * 16 * DH
PPC = (GPC * 16) // RPP
NUNITS = 2 * NBINS
NWORKERS = 32
UNITS_PER_TILE = 4
QCHUNK_WORDS = NUNITS * PIECE_WORDS
NQ = NROI_PAD // RPP
NQ_T = 157
QPT = 5
TAIL_ROIS = NROI - (NQ_T - 1) * RPP


def _integral_tc_kernel(f_ref, o_ref):
    row = lax.broadcasted_iota(jnp.int32, (H, H), 0)
    col = lax.broadcasted_iota(jnp.int32, (H, H), 1)
    lower = (row >= col).astype(jnp.float32)
    upper = (row <= col).astype(jnp.float32)
    for d in range(D):
        f = f_ref[0, d, 0]
        a = jnp.dot(lower, f, preferred_element_type=jnp.float32,
                    precision=lax.Precision.HIGHEST)
        b = jnp.dot(a, upper, preferred_element_type=jnp.float32,
                    precision=lax.Precision.HIGHEST)
        buf = jnp.concatenate(
            [jnp.zeros((H, 1), jnp.float32), b,
             jnp.zeros((H, WPAD - 1 - W), jnp.float32)], axis=1)
        buf = jnp.concatenate([jnp.zeros((1, WPAD), jnp.float32), buf],
                              axis=0)
        o_ref[0, 0, d] = buf
    o_ref[0, 0, D] = jnp.zeros((H + 1, WPAD), jnp.float32)


def _integral_image(feat5):
    return pl.pallas_call(
        _integral_tc_kernel,
        grid=(2, NBINS),
        in_specs=[pl.BlockSpec((1, D, 1, H, W), lambda b, p: (b, 0, p, 0, 0))],
        out_specs=pl.BlockSpec((1, 1, DPAD, H + 1, WPAD),
                               lambda b, p: (b, p, 0, 0, 0)),
        out_shape=jax.ShapeDtypeStruct((2, NBINS, DPAD, H + 1, WPAD),
                                       jnp.float32),
    )(feat5)


def _bin_bounds(rois):
    pf = jnp.arange(G, dtype=jnp.float32)[None, :]
    rsw = (jnp.round(rois[:, 1]) * SCALE)[:, None]
    rsh = (jnp.round(rois[:, 2]) * SCALE)[:, None]
    rew = (jnp.round(rois[:, 3] + 1.0) * SCALE)[:, None]
    reh = (jnp.round(rois[:, 4] + 1.0) * SCALE)[:, None]
    roi_w = jnp.maximum(rew - rsw, 0.1)
    roi_h = jnp.maximum(reh - rsh, 0.1)
    bsh = roi_h / G
    bsw = roi_w / G
    hs = jnp.clip(jnp.floor(pf * bsh + rsh), 0, H).astype(jnp.int32)
    he = jnp.clip(jnp.ceil((pf + 1.0) * bsh + rsh), 0, H).astype(jnp.int32)
    ws = jnp.clip(jnp.floor(pf * bsw + rsw), 0, W).astype(jnp.int32)
    we = jnp.clip(jnp.ceil((pf + 1.0) * bsw + rsw), 0, W).astype(jnp.int32)
    return hs, he, ws, we


def _pool_sc_kernel(i_hbm, hshe_hbm, wswe_hbm, batch_hbm,
                    out_hbm, unit_v, hshe_v, wswe_v, batch_v, stage_v, sem):
    wid = lax.axis_index("s") * 2 + lax.axis_index("c")
    lanes = lax.iota(jnp.int32, 16)
    lanes_d = lanes * DH
    pltpu.sync_copy(batch_hbm, batch_v)

    def process_unit(u):
        binidx = u // 2
        dh = u - binidx * 2
        ph = binidx // G
        pw = binidx - ph * G
        doff = dh * HALF_WORDS
        for b in range(2):
            src = (b * NBINS + binidx) * PLANE_WORDS + doff
            pltpu.sync_copy(
                i_hbm.at[pl.ds(src, HALF_WORDS)],
                unit_v.at[pl.ds(b * HALF_WORDS, HALF_WORDS)])
        pltpu.sync_copy(hshe_hbm.at[ph], hshe_v)
        pltpu.sync_copy(wswe_hbm.at[pw], wswe_v)
        unit_off = binidx * (2 * PIECE_WORDS) + dh * PIECE_WORDS

        def chunk_body(c, carry):
            @plsc.parallel_loop(0, GPC, unroll=2)
            def group_body(j):
                base = (c * GPC + j) * 16
                xh = hshe_v[pl.ds(base, 16)]
                xw = wswe_v[pl.ds(base, 16)]
                bvec = batch_v[pl.ds(base, 16)]
                hs = xh & 0xFFFF
                he = xh >> 16
                ws = xw & 0xFFFF
                we = xw >> 16
                area = ((he - hs) * (we - ws)).astype(jnp.float32)
                empty = (he <= hs) | (we <= ws)
                inv = jnp.where(empty, 0.0, 1.0 / jnp.maximum(area, 1.0))
                bterm = bvec * HALF_WORDS
                top = hs * WPAD + bterm
                bot = he * WPAD + bterm
                i_ee = bot + we
                i_se = top + we
                i_es = bot + ws
                i_ss = top + ws
                sbase = j * (16 * DH) + lanes_d
                for d in range(DH):
                    sub = unit_v.at[pl.ds(d * PLANE_HW,
                                          UNIT_WORDS - d * PLANE_HW)]
                    g1 = plsc.load_gather(sub, [i_ee])
                    g2 = plsc.load_gather(sub, [i_se])
                    g3 = plsc.load_gather(sub, [i_es])
                    g4 = plsc.load_gather(sub, [i_ss])
                    val = (g1 - g2 - g3 + g4) * inv
                    plsc.store_scatter(stage_v, [sbase + d], val)

            copies = []
            for i in range(PPC):
                q = c * PPC + i
                copies.append(pltpu.async_copy(
                    stage_v.at[pl.ds(i * PIECE_WORDS, PIECE_WORDS)],
                    out_hbm.at[pl.ds(q * QCHUNK_WORDS + unit_off,
                                     PIECE_WORDS)],
                    sem))
            for cp in copies:
                cp.wait()
            return carry

        lax.fori_loop(0, NCHUNKS, chunk_body, 0)

    def unit_body(k, carry):
        u = wid + k * NWORKERS

        @pl.when(u < NUNITS)
        def _():
            process_unit(u)

        return carry

    lax.fori_loop(0, UNITS_PER_TILE, unit_body, 0)


def _pool(i_img, hshe, wswe, batch):
    mesh = plsc.VectorSubcoreMesh(core_axis_name="c", subcore_axis_name="s")
    f = pl.kernel(
        _pool_sc_kernel,
        out_type=jax.ShapeDtypeStruct((NQ * QCHUNK_WORDS,), jnp.float32),
        mesh=mesh,
        compiler_params=pltpu.CompilerParams(needs_layout_passes=False),
        scratch_types=[
            pltpu.VMEM((UNIT_WORDS,), jnp.float32),
            pltpu.VMEM((NROI_PAD,), jnp.int32),
            pltpu.VMEM((NROI_PAD,), jnp.int32),
            pltpu.VMEM((NROI_PAD,), jnp.int32),
            pltpu.VMEM((STAGE_WORDS,), jnp.float32),
            pltpu.SemaphoreType.DMA,
        ],
    )
    return f(i_img, hshe, wswe, batch)


def _transpose_sc_kernel(pool_hbm, out_hbm, in_v, obuf_v, sem):
    wid = lax.axis_index("s") * 2 + lax.axis_index("c")
    lanes = lax.iota(jnp.int32, 16)

    def process_chunk(q):
        pltpu.sync_copy(pool_hbm.at[pl.ds(q * QCHUNK_WORDS, QCHUNK_WORDS)],
                        in_v)
        boffs = []
        osels = []
        for half in range(RPP // 16):
            r_vec = lanes + half * 16
            boffs.append(r_vec * DH)
            osels.append(r_vec)

        @plsc.parallel_loop(0, NC, unroll=4)
        def c_body(c):
            binidx = c % NBINS
            d = c // NBINS
            dh = d // DH
            dl = d - dh * DH
            pre_c = binidx * (2 * PIECE_WORDS) + dh * PIECE_WORDS + dl
            cvec = jnp.full((16,), c, jnp.int32)
            for half in range(RPP // 16):
                v = plsc.load_gather(in_v, [boffs[half] + pre_c])
                plsc.store_scatter(obuf_v, [osels[half], cvec], v)

        @pl.when(q < NQ_T - 1)
        def _():
            pltpu.sync_copy(obuf_v, out_hbm.at[pl.ds(q * RPP, RPP)])

        @pl.when(q == NQ_T - 1)
        def _():
            pltpu.sync_copy(
                obuf_v.at[pl.ds(0, TAIL_ROIS)],
                out_hbm.at[pl.ds(q * RPP, TAIL_ROIS)])

    def chunk_loop(k, carry):
        q = wid + k * NWORKERS

        @pl.when(q < NQ_T)
        def _():
            process_chunk(q)

        return carry

    lax.fori_loop(0, QPT, chunk_loop, 0)


def _transpose(pool_out):
    mesh = plsc.VectorSubcoreMesh(core_axis_name="c", subcore_axis_name="s")
    f = pl.kernel(
        _transpose_sc_kernel,
        out_type=jax.ShapeDtypeStruct((NROI, NC), jnp.float32),
        mesh=mesh,
        compiler_params=pltpu.CompilerParams(needs_layout_passes=False),
        scratch_types=[
            pltpu.VMEM((QCHUNK_WORDS,), jnp.float32),
            pltpu.VMEM((RPP, NC), jnp.float32),
            pltpu.SemaphoreType.DMA,
        ],
    )
    return f(pool_out)


def kernel(features, rois):
    feat5 = features.reshape(2, D, NBINS, H, W)
    i_img = _integral_image(feat5).reshape(NUNITS * PLANE_WORDS)
    hs, he, ws, we = _bin_bounds(rois)
    pad_b = jnp.zeros((NROI_PAD - NROI, G), jnp.int32)

    def padt(x):
        return jnp.concatenate([x, pad_b], axis=0).T

    hshe = padt(hs | (he << 16))
    wswe = padt(ws | (we << 16))
    batch = jnp.concatenate(
        [rois[:, 0].astype(jnp.int32),
         jnp.zeros((NROI_PAD - NROI,), jnp.int32)])
    pool_out = _pool(i_img, hshe, wswe, batch)
    return _transpose(pool_out).reshape(NROI, D, G, G)
